# baseline (speedup 1.0000x reference)
import functools

import jax
import jax.numpy as jnp
from jax import lax
from jax.experimental import pallas as pl
from jax.experimental.pallas import tpu as pltpu
from jax.experimental.pallas import tpu_sc as plsc

B = 8
CIN = 3
HW = 512
C = 512
P = 16
K = 1024
D = 256
S = 1024
F = 768
NJ = 4
N_ROWS = B * C * NJ
LOSS_SCALE = 1.0 / float(B * S * C // 2)


def _enc_vq_body(xp_ref, we_ref, be_ref, cbt_ref, idx_ref, loss_ref):
    b = pl.program_id(0)
    j = pl.program_id(1)
    xr = xp_ref[0].reshape(CIN, 8, P, 32, P)
    xj = xr.transpose(0, 2, 4, 1, 3).reshape(F, D)
    zj = jnp.dot(we_ref[...], xj, preferred_element_type=jnp.float32)
    zj = zj + be_ref[...]
    dots = jnp.dot(zj, cbt_ref[...], preferred_element_type=jnp.float32)
    x2 = jnp.sum(zj * zj, axis=1, keepdims=True)
    e2 = jnp.sum(cbt_ref[...] * cbt_ref[...], axis=0, keepdims=True)
    dist = x2 - 2.0 * dots + e2
    mval = jnp.min(dist, axis=1, keepdims=True)
    iota = lax.broadcasted_iota(jnp.int32, dist.shape, 1)
    idxj = jnp.min(jnp.where(dist <= mval, iota, jnp.int32(2**30)),
                   axis=1, keepdims=True)
    idx_ref[0] = idxj

    @pl.when((b == 0) & (j == 0))
    def _init():
        loss_ref[0, 0] = 0.0

    loss_ref[0, 0] += jnp.sum(mval) * LOSS_SCALE


def _dec_body(zq_ref, wd_ref, bd_ref, out_ref):
    m = (jnp.dot(wd_ref[...], zq_ref[0, 0],
                 preferred_element_type=jnp.float32) + bd_ref[...])
    m = m.reshape(P, P, CIN, 8, 32)
    out_ref[0] = m.transpose(2, 3, 0, 4, 1).reshape(CIN, 128, HW)


_CHUNK = 256


@functools.cache
def _make_sc_gather():
    info = plsc.get_sparse_core_info()
    nc, ns = info.num_cores, info.num_subcores
    rows_per_w = N_ROWS // (nc * ns)

    @functools.partial(
        pl.kernel,
        out_type=jax.ShapeDtypeStruct((N_ROWS, D), jnp.float32),
        mesh=plsc.VectorSubcoreMesh(core_axis_name="c", subcore_axis_name="s"),
        scratch_types=[
            pltpu.VMEM((_CHUNK,), jnp.int32),
            pltpu.VMEM((_CHUNK, D), jnp.float32),
            pltpu.SemaphoreType.DMA,
        ],
    )
    def _sc_gather(cb_hbm, idx_hbm, out_hbm, idx_v, rows_v, sem):
        wid = lax.axis_index("s") * nc + lax.axis_index("c")
        base = wid * rows_per_w
        for t in range(rows_per_w // _CHUNK):
            off = base + t * _CHUNK
            pltpu.sync_copy(idx_hbm.at[pl.ds(off, _CHUNK)], idx_v)
            pltpu.async_copy(cb_hbm.at[idx_v], rows_v, sem).wait()
            pltpu.sync_copy(rows_v, out_hbm.at[pl.ds(off, _CHUNK)])

    return _sc_gather


def kernel(X, W_enc, b_enc, codebook, W_dec, b_dec):
    We = W_enc.reshape(C, F)
    cbT = codebook.T
    be = b_enc[:, None]
    Wd = W_dec[::-1, ::-1].transpose(0, 1, 3, 2).reshape(F, C)
    bd = jnp.tile(b_dec, F // CIN)[:, None]

    idx, loss = pl.pallas_call(
        _enc_vq_body,
        grid=(B, NJ),
        in_specs=[
            pl.BlockSpec((1, CIN, 128, HW), lambda b, j: (b, 0, j, 0)),
            pl.BlockSpec((C, F), lambda b, j: (0, 0)),
            pl.BlockSpec((C, 1), lambda b, j: (0, 0)),
            pl.BlockSpec((D, K), lambda b, j: (0, 0)),
        ],
        out_specs=[
            pl.BlockSpec((1, C, 1), lambda b, j: (b * NJ + j, 0, 0)),
            pl.BlockSpec(memory_space=pltpu.SMEM, block_shape=(1, 1),
                         index_map=lambda b, j: (0, 0)),
        ],
        out_shape=[
            jax.ShapeDtypeStruct((B * NJ, C, 1), jnp.int32),
            jax.ShapeDtypeStruct((1, 1), jnp.float32),
        ],
    )(X, We, be, cbT)

    zq = _make_sc_gather()(codebook, idx.reshape(N_ROWS))
    zq = zq.reshape(B, NJ, C, D)

    out = pl.pallas_call(
        _dec_body,
        grid=(B, NJ),
        in_specs=[
            pl.BlockSpec((1, 1, C, D), lambda b, j: (b, j, 0, 0)),
            pl.BlockSpec((F, C), lambda b, j: (0, 0)),
            pl.BlockSpec((F, 1), lambda b, j: (0, 0)),
        ],
        out_specs=pl.BlockSpec((1, CIN, 128, HW), lambda b, j: (b, 0, j, 0)),
        out_shape=jax.ShapeDtypeStruct((B, CIN, HW, HW), jnp.float32),
    )(zq, Wd, bd)

    return out, loss[0, 0]
